# baseline (speedup 1.0000x reference)
import functools

import jax
import jax.numpy as jnp
from jax import lax
from jax.experimental import pallas as pl
from jax.experimental.pallas import tpu as pltpu
from jax.experimental.pallas import tpu_sc as plsc

_NUM_CORES = 2
_NUM_SUBCORES = 16
_NW = _NUM_CORES * _NUM_SUBCORES
_LANES = 16

_B, _T = 16384, 200
_C, _D = 256, 8
_N = _B * _T
_PER_W = _N // _NW
_CHUNK = 6400
_NCHUNK = _PER_W // _CHUNK
_GROUPS = _CHUNK // _LANES


def _make_lookup():
    mesh = plsc.VectorSubcoreMesh(core_axis_name="c", subcore_axis_name="s")

    @functools.partial(
        pl.kernel,
        out_type=jax.ShapeDtypeStruct((_N * _D,), jnp.float32),
        mesh=mesh,
        scratch_types=[
            pltpu.VMEM((_C * _D,), jnp.float32),
            pltpu.VMEM((_CHUNK,), jnp.int32),
            pltpu.VMEM((_CHUNK,), jnp.int32),
            pltpu.VMEM((_CHUNK * _D,), jnp.float32),
            pltpu.VMEM((_CHUNK * _D,), jnp.float32),
            pltpu.SemaphoreType.DMA,
            pltpu.SemaphoreType.DMA,
            pltpu.SemaphoreType.DMA,
            pltpu.SemaphoreType.DMA,
        ],
        compiler_params=pltpu.CompilerParams(needs_layout_passes=False),
    )
    def lookup(idx_hbm, table_hbm, out_hbm, table_v, idx_v0, idx_v1,
               out_v0, out_v1, si0, si1, so0, so1):
        wid = lax.axis_index("s") * _NUM_CORES + lax.axis_index("c")
        pltpu.sync_copy(table_hbm, table_v)
        pos0 = lax.iota(jnp.int32, _LANES) * _D
        idx_bufs, out_bufs = (idx_v0, idx_v1), (out_v0, out_v1)
        isems, osems = (si0, si1), (so0, so1)
        w0 = wid * _PER_W

        def start_idx(c):
            return pltpu.async_copy(
                idx_hbm.at[pl.ds(w0 + c * _CHUNK, _CHUNK)],
                idx_bufs[c % 2], isems[c % 2])

        d_idx, d_out = {}, {}
        d_idx[0] = start_idx(0)
        for c in range(_NCHUNK):
            p = c % 2
            if c + 1 < _NCHUNK:
                d_idx[c + 1] = start_idx(c + 1)
            d_idx[c].wait()
            if c >= 2:
                d_out[c - 2].wait()

            @pl.loop(0, _GROUPS)
            def _group(g, idx_v=idx_bufs[p], out_v=out_bufs[p]):
                i16 = idx_v[pl.ds(g * _LANES, _LANES)]
                gidx0 = i16 * _D
                obase = pos0 + g * (_LANES * _D)
                for j in range(_D):
                    vals = plsc.load_gather(table_v, [gidx0 + j])
                    plsc.store_scatter(out_v, [obase + j], vals)

            d_out[c] = pltpu.async_copy(
                out_bufs[p],
                out_hbm.at[pl.ds((w0 + c * _CHUNK) * _D, _CHUNK * _D)],
                osems[p])
        d_out[_NCHUNK - 2].wait()
        d_out[_NCHUNK - 1].wait()

    return lookup


_lookup = _make_lookup()


def kernel(idx, codebook):
    b, t = idx.shape
    _, d = codebook.shape
    out = _lookup(idx.reshape(-1), codebook.reshape(-1))
    return out.reshape(b, t, d)


# speedup vs baseline: 1.1226x; 1.1226x over previous
import functools

import jax
import jax.numpy as jnp
from jax import lax
from jax.experimental import pallas as pl
from jax.experimental.pallas import tpu as pltpu
from jax.experimental.pallas import tpu_sc as plsc

_NUM_CORES = 2
_NUM_SUBCORES = 16
_NW = _NUM_CORES * _NUM_SUBCORES
_LANES = 16

_B, _T = 16384, 200
_C, _D = 256, 8
_N = _B * _T
_PER_W = _N // _NW
_CHUNK = 6400
_NCHUNK = _PER_W // _CHUNK
_GROUPS = _CHUNK // _LANES


def _make_lookup():
    mesh = plsc.VectorSubcoreMesh(core_axis_name="c", subcore_axis_name="s")

    @functools.partial(
        pl.kernel,
        out_type=jax.ShapeDtypeStruct((_N * _D,), jnp.float32),
        mesh=mesh,
        scratch_types=[
            pltpu.VMEM((_C * _D,), jnp.float32),
            pltpu.VMEM((_CHUNK,), jnp.int32),
            pltpu.VMEM((_CHUNK,), jnp.int32),
            pltpu.VMEM((_CHUNK * _D,), jnp.float32),
            pltpu.VMEM((_CHUNK * _D,), jnp.float32),
            pltpu.SemaphoreType.DMA,
            pltpu.SemaphoreType.DMA,
            pltpu.SemaphoreType.DMA,
            pltpu.SemaphoreType.DMA,
        ],
        compiler_params=pltpu.CompilerParams(needs_layout_passes=False),
    )
    def lookup(idx_hbm, table_hbm, out_hbm, table_v, idx_v0, idx_v1,
               out_v0, out_v1, si0, si1, so0, so1):
        wid = lax.axis_index("s") * _NUM_CORES + lax.axis_index("c")
        pltpu.sync_copy(table_hbm, table_v)
        pos0 = lax.iota(jnp.int32, _LANES) * _D
        idx_bufs, out_bufs = (idx_v0, idx_v1), (out_v0, out_v1)
        isems, osems = (si0, si1), (so0, so1)
        w0 = wid * _PER_W

        def start_idx(c):
            return pltpu.async_copy(
                idx_hbm.at[pl.ds(w0 + c * _CHUNK, _CHUNK)],
                idx_bufs[c % 2], isems[c % 2])

        d_idx, d_out = {}, {}
        d_idx[0] = start_idx(0)
        for c in range(_NCHUNK):
            p = c % 2
            if c + 1 < _NCHUNK:
                d_idx[c + 1] = start_idx(c + 1)
            d_idx[c].wait()
            if c >= 2:
                d_out[c - 2].wait()

            @plsc.parallel_loop(0, _GROUPS, unroll=4)
            def _group(g, idx_v=idx_bufs[p], out_v=out_bufs[p]):
                i16 = idx_v[pl.ds(g * _LANES, _LANES)]
                gidx0 = i16 * _D
                obase = pos0 + g * (_LANES * _D)
                for j in range(_D):
                    vals = plsc.load_gather(table_v, [gidx0 + j])
                    plsc.store_scatter(out_v, [obase + j], vals)

            d_out[c] = pltpu.async_copy(
                out_bufs[p],
                out_hbm.at[pl.ds((w0 + c * _CHUNK) * _D, _CHUNK * _D)],
                osems[p])
        d_out[_NCHUNK - 2].wait()
        d_out[_NCHUNK - 1].wait()

    return lookup


_lookup = _make_lookup()


def kernel(idx, codebook):
    b, t = idx.shape
    _, d = codebook.shape
    out = _lookup(idx.reshape(-1), codebook.reshape(-1))
    return out.reshape(b, t, d)


# speedup vs baseline: 28.6106x; 25.4855x over previous
import functools

import jax
import jax.numpy as jnp
from jax import lax
from jax.experimental import pallas as pl
from jax.experimental.pallas import tpu as pltpu
from jax.experimental.pallas import tpu_sc as plsc

_NUM_CORES = 2
_NUM_SUBCORES = 16
_NW = _NUM_CORES * _NUM_SUBCORES
_LANES = 16

_B, _T = 16384, 200
_C, _D = 256, 8
_N = _B * _T

_TT = _T // 8
_BT = _B // 128
_BTC = 4
_NITEMS = _TT * (_BT // _BTC)
_PER_W = _NITEMS // _NW
_ITEM_IDX = _BTC * 8 * 128
_GROUPS = _ITEM_IDX // _LANES
_ITEM_OUT = _ITEM_IDX * _D
_TS_OUT = _BTC * _D * 128


def _make_lookup():
    mesh = plsc.VectorSubcoreMesh(core_axis_name="c", subcore_axis_name="s")

    @functools.partial(
        pl.kernel,
        out_type=jax.ShapeDtypeStruct((_N * _D,), jnp.float32),
        mesh=mesh,
        scratch_types=[
            pltpu.VMEM((_C * _D,), jnp.float32),
            pltpu.VMEM((_ITEM_IDX,), jnp.int32),
            pltpu.VMEM((_ITEM_IDX,), jnp.int32),
            pltpu.VMEM((_ITEM_OUT,), jnp.float32),
            pltpu.VMEM((_ITEM_OUT,), jnp.float32),
            pltpu.SemaphoreType.DMA,
            pltpu.SemaphoreType.DMA,
            pltpu.SemaphoreType.DMA,
            pltpu.SemaphoreType.DMA,
        ],
        compiler_params=pltpu.CompilerParams(needs_layout_passes=False),
    )
    def lookup(idx_hbm, table_hbm, out_hbm, table_v, idx_v0, idx_v1,
               out_v0, out_v1, si0, si1, so0, so1):
        wid = lax.axis_index("s") * _NUM_CORES + lax.axis_index("c")
        pltpu.sync_copy(table_hbm, table_v)
        lane = lax.iota(jnp.int32, _LANES)
        idx_bufs, out_bufs = (idx_v0, idx_v1), (out_v0, out_v1)
        isems, osems = (si0, si1), (so0, so1)

        def item_of(k):
            return wid + k * _NW

        def start_idx(k):
            item = item_of(k)
            tt = item // (_BT // _BTC)
            btc = item % (_BT // _BTC)
            base = (tt * _BT + btc * _BTC) * 8 * 128
            return pltpu.async_copy(
                idx_hbm.at[pl.ds(base, _ITEM_IDX)],
                idx_bufs[k % 2], isems[k % 2])

        def start_out(k):
            item = item_of(k)
            tt = item // (_BT // _BTC)
            btc = item % (_BT // _BTC)
            descs = []
            for ts in range(8):
                base = ((tt * 8 + ts) * _BT + btc * _BTC) * _D * 128
                descs.append(pltpu.async_copy(
                    out_bufs[k % 2].at[pl.ds(ts * _TS_OUT, _TS_OUT)],
                    out_hbm.at[pl.ds(base, _TS_OUT)],
                    osems[k % 2]))
            return descs

        d_idx, d_out = {}, {}
        d_idx[0] = start_idx(0)
        for k in range(_PER_W):
            p = k % 2
            if k + 1 < _PER_W:
                d_idx[k + 1] = start_idx(k + 1)
            d_idx[k].wait()
            if k >= 2:
                for d in d_out[k - 2]:
                    d.wait()

            @plsc.parallel_loop(0, _GROUPS, unroll=2)
            def _group(g, idx_v=idx_bufs[p], out_v=out_bufs[p]):
                bl0 = (g & 7) * _LANES
                ts = (g >> 3) & 7
                bt_rel = g >> 6
                obase = ts * _TS_OUT + bt_rel * (_D * 128) + bl0
                i16 = idx_v[pl.ds(g * _LANES, _LANES)]
                ovec = lane + obase
                for j in range(_D):
                    vals = plsc.load_gather(table_v, [i16 + j * _C])
                    plsc.store_scatter(out_v, [ovec + j * 128], vals)

            d_out[k] = start_out(k)
        for d in d_out[_PER_W - 2]:
            d.wait()
        for d in d_out[_PER_W - 1]:
            d.wait()

    return lookup


_lookup = _make_lookup()


def kernel(idx, codebook):
    idx_raw = (idx.reshape(_BT, 128, _TT, 8)
               .transpose(2, 0, 3, 1)
               .reshape(-1))
    table_jc = jnp.transpose(codebook).reshape(-1)
    out_raw = _lookup(idx_raw, table_jc)
    return (out_raw.reshape(_T, _BT, _D, 128)
            .transpose(1, 3, 0, 2)
            .reshape(_B, _T, _D))
